# Initial kernel scaffold; baseline (speedup 1.0000x reference)
import jax
import jax.numpy as jnp
from jax.experimental import pallas as pl


def kernel(xyz, new_xyz, features):
    raise NotImplementedError("write your pallas kernel here")


# speedup vs baseline: 11.5706x; 11.5706x over previous
import functools
import jax
import jax.numpy as jnp
from jax import lax
from jax.experimental import pallas as pl
from jax.experimental.pallas import tpu as pltpu
from jax.experimental.pallas import tpu_sc as plsc

RADIUS = 0.12
NSAMPLE = 32

B, N, NP, C = 4, 8192, 2048, 64
QB = 256
K = 512
NCHUNK = N // K
COUT = C + 3


def _ball_query_body(xyz_ref, q_ref, tri_ref, cnt_ref, idx_ref):
    q = q_ref[0]
    qq = jnp.sum(q * q, axis=1, keepdims=True)
    r2 = RADIUS * RADIUS

    def chunk(k, carry):
        acc, cnt = carry
        xc = xyz_ref[0, pl.ds(k * K, K), :]
        pp = jnp.sum(xc * xc, axis=1)[None, :]
        qp = jnp.dot(q, xc.T, preferred_element_type=jnp.float32)
        d2 = qq + pp - 2.0 * qp
        mb = d2 < r2
        m = mb.astype(jnp.float32)
        cs = jnp.dot(m, tri_ref[...], preferred_element_type=jnp.float32)
        slot = cnt + cs
        gidx = (jnp.float32(k * K)
                + lax.broadcasted_iota(jnp.int32, (1, K), 1).astype(jnp.float32))
        cols = []
        for s in range(NSAMPLE):
            sel = jnp.where(mb & (slot == jnp.float32(s)), gidx, 0.0)
            cols.append(jnp.sum(sel, axis=1, keepdims=True))
        acc = acc + jnp.concatenate(cols, axis=1)
        cnt = cnt + jnp.sum(m, axis=1, keepdims=True)
        return acc, cnt

    acc0 = jnp.zeros((QB, NSAMPLE), jnp.float32)
    cnt0 = jnp.zeros((QB, 1), jnp.float32)
    acc, cnt = lax.fori_loop(0, NCHUNK, chunk, (acc0, cnt0))

    cnt_c = jnp.minimum(cnt, jnp.float32(NSAMPLE))
    s_iota = lax.broadcasted_iota(jnp.int32, (QB, NSAMPLE), 1).astype(jnp.float32)
    first = acc[:, 0:1]
    idx = jnp.where(s_iota < cnt_c, acc, first)
    cnt_ref[0, 0, :] = cnt_c[:, 0]
    idx_ref[0] = idx.astype(jnp.int32)


def _ball_query(xyz, new_xyz, tri):
    grid = (B, NP // QB)
    return pl.pallas_call(
        _ball_query_body,
        grid=grid,
        in_specs=[
            pl.BlockSpec((1, N, 3), lambda b, q: (b, 0, 0)),
            pl.BlockSpec((1, QB, 3), lambda b, q: (b, q, 0)),
            pl.BlockSpec((K, K), lambda b, q: (0, 0)),
        ],
        out_specs=[
            pl.BlockSpec((1, 1, QB), lambda b, q: (b, 0, q)),
            pl.BlockSpec((1, QB, NSAMPLE), lambda b, q: (b, q, 0)),
        ],
        out_shape=[
            jax.ShapeDtypeStruct((B, 1, NP), jnp.float32),
            jax.ShapeDtypeStruct((B, NP, NSAMPLE), jnp.int32),
        ],
    )(xyz, new_xyz, tri)


NW = 32
CG = 8
QC = 128
FLAT = QC * NSAMPLE
NVR = FLAT // 16
NQC = NP // QC


def _gather_body(xyzt_hbm, feat_hbm, nq_hbm, idx_hbm, oxyz_hbm, ofeat_hbm,
                 feat_v, idx_v, out_v, nq_v):
    wid = lax.axis_index("c") * 16 + lax.axis_index("s")

    b1 = wid // 8
    fg1 = wid % 8

    def run_feat():
        c0 = fg1 * CG
        pltpu.sync_copy(feat_hbm.at[b1, 0, pl.ds(c0 * N, CG * N)], feat_v)

        def qchunk(qc, _):
            pltpu.sync_copy(idx_hbm.at[b1, 0, pl.ds(qc * FLAT, FLAT)], idx_v)
            for c in range(CG):
                coff = jnp.full((16,), c * N, jnp.int32)

                def vloop(v, _):
                    iv = idx_v[pl.ds(v * 16, 16)]
                    g = plsc.load_gather(feat_v, [iv + coff])
                    out_v[c, pl.ds(v * 16, 16)] = g
                    return 0

                lax.fori_loop(0, NVR, vloop, 0)
            pltpu.sync_copy(
                out_v, ofeat_hbm.at[b1, pl.ds(c0, CG), pl.ds(qc * FLAT, FLAT)])
            return 0

        lax.fori_loop(0, NQC, qchunk, 0)

    run_feat()

    @pl.when(wid < 8)
    def _():
        b2 = wid // 2
        half = wid % 2
        pltpu.sync_copy(xyzt_hbm.at[b2, 0, :], feat_v.at[pl.ds(0, 3 * N)])

        def qchunk(qc, _):
            pltpu.sync_copy(idx_hbm.at[b2, 0, pl.ds(qc * FLAT, FLAT)], idx_v)
            pltpu.sync_copy(nq_hbm.at[b2, 0, pl.ds(qc * QC * 3, QC * 3)], nq_v)
            for c in range(3):
                coff = jnp.full((16,), c * N, jnp.int32)

                def vloop(v, _):
                    iv = idx_v[pl.ds(v * 16, 16)]
                    g = plsc.load_gather(feat_v, [iv + coff])
                    nqi = jnp.broadcast_to((v // 2) * 3 + c, (16,))
                    g = g - plsc.load_gather(nq_v, [nqi])
                    out_v[c, pl.ds(v * 16, 16)] = g
                    return 0

                lax.fori_loop(0, NVR, vloop, 0)
            pltpu.sync_copy(
                out_v.at[pl.ds(0, 3), :],
                oxyz_hbm.at[b2, :, pl.ds(qc * FLAT, FLAT)])
            return 0

        lax.fori_loop(half * (NQC // 2), (half + 1) * (NQC // 2), qchunk, 0)


def _gather(xyzt, features, new_xyz, idx_flat):
    mesh = plsc.VectorSubcoreMesh(core_axis_name="c", subcore_axis_name="s",
                                  num_cores=2, num_subcores=16)
    f = pl.kernel(
        _gather_body,
        out_type=[
            jax.ShapeDtypeStruct((B, 3, NP * NSAMPLE), jnp.float32),
            jax.ShapeDtypeStruct((B, C, NP * NSAMPLE), jnp.float32),
        ],
        mesh=mesh,
        scratch_types=[
            pltpu.VMEM((CG * N,), jnp.float32),
            pltpu.VMEM((FLAT,), jnp.int32),
            pltpu.VMEM((CG, FLAT), jnp.float32),
            pltpu.VMEM((QC * 3,), jnp.float32),
        ],
        compiler_params=pltpu.CompilerParams(needs_layout_passes=False),
    )
    return f(xyzt, features, new_xyz, idx_flat)


def kernel(xyz, new_xyz, features):
    tri = jnp.triu(jnp.ones((K, K), jnp.float32), k=1)
    cnt_f, idx = _ball_query(xyz, new_xyz, tri)
    xyzt = jnp.transpose(xyz, (0, 2, 1)).reshape(B, 1, 3 * N)
    feats_flat = features.reshape(B, 1, C * N)
    nq_flat = new_xyz.reshape(B, 1, NP * 3)
    idx_flat = idx.reshape(B, 1, NP * NSAMPLE)
    oxyz, ofeat = _gather(xyzt, feats_flat, nq_flat, idx_flat)
    new_features = jnp.concatenate(
        [oxyz.reshape(B, 3, NP, NSAMPLE), ofeat.reshape(B, C, NP, NSAMPLE)],
        axis=1)
    return cnt_f.reshape(B, NP).astype(jnp.int32), new_features


# speedup vs baseline: 13.5241x; 1.1688x over previous
import functools
import jax
import jax.numpy as jnp
from jax import lax
from jax.experimental import pallas as pl
from jax.experimental.pallas import tpu as pltpu
from jax.experimental.pallas import tpu_sc as plsc

RADIUS = 0.12
NSAMPLE = 32

B, N, NP, C = 4, 8192, 2048, 64
QB = 256
K = 512
NCHUNK = N // K
COUT = C + 3


def _ball_query_body(xyz_ref, q_ref, tri_ref, cnt_ref, idx_ref):
    q = q_ref[0]
    qq = jnp.sum(q * q, axis=1, keepdims=True)
    r2 = RADIUS * RADIUS

    def chunk(k, carry):
        acc, cnt = carry
        xc = xyz_ref[0, pl.ds(k * K, K), :]
        pp = jnp.sum(xc * xc, axis=1)[None, :]
        qp = jnp.dot(q, xc.T, preferred_element_type=jnp.float32)
        d2 = qq + pp - 2.0 * qp
        mb = d2 < r2
        m = mb.astype(jnp.float32)
        cs = jnp.dot(m, tri_ref[...], preferred_element_type=jnp.float32)
        slot = cnt + cs
        gidx = (jnp.float32(k * K)
                + lax.broadcasted_iota(jnp.int32, (1, K), 1).astype(jnp.float32))
        cols = []
        for s in range(NSAMPLE):
            sel = jnp.where(mb & (slot == jnp.float32(s)), gidx, 0.0)
            cols.append(jnp.sum(sel, axis=1, keepdims=True))
        acc = acc + jnp.concatenate(cols, axis=1)
        cnt = cnt + jnp.sum(m, axis=1, keepdims=True)
        return acc, cnt

    acc0 = jnp.zeros((QB, NSAMPLE), jnp.float32)
    cnt0 = jnp.zeros((QB, 1), jnp.float32)
    acc, cnt = lax.fori_loop(0, NCHUNK, chunk, (acc0, cnt0))

    cnt_c = jnp.minimum(cnt, jnp.float32(NSAMPLE))
    s_iota = lax.broadcasted_iota(jnp.int32, (QB, NSAMPLE), 1).astype(jnp.float32)
    first = acc[:, 0:1]
    idx = jnp.where(s_iota < cnt_c, acc, first)
    cnt_ref[0, 0, :] = cnt_c[:, 0]
    idx_ref[0] = idx.astype(jnp.int32)


def _ball_query(xyz, new_xyz, tri):
    grid = (B, NP // QB)
    return pl.pallas_call(
        _ball_query_body,
        grid=grid,
        in_specs=[
            pl.BlockSpec((1, N, 3), lambda b, q: (b, 0, 0)),
            pl.BlockSpec((1, QB, 3), lambda b, q: (b, q, 0)),
            pl.BlockSpec((K, K), lambda b, q: (0, 0)),
        ],
        out_specs=[
            pl.BlockSpec((1, 1, QB), lambda b, q: (b, 0, q)),
            pl.BlockSpec((1, QB, NSAMPLE), lambda b, q: (b, q, 0)),
        ],
        out_shape=[
            jax.ShapeDtypeStruct((B, 1, NP), jnp.float32),
            jax.ShapeDtypeStruct((B, NP, NSAMPLE), jnp.int32),
        ],
    )(xyz, new_xyz, tri)


NW = 32
CG = 8
QC = 128
FLAT = QC * NSAMPLE
NVR = FLAT // 16
NQC = NP // QC


def _gather_body(xyzt_hbm, feat_hbm, nq_hbm, idx_hbm, oxyz_hbm, ofeat_hbm,
                 feat_v, idx_v, out_v, nq_v):
    wid = lax.axis_index("c") * 16 + lax.axis_index("s")

    b1 = wid // 8
    fg1 = wid % 8

    def run_feat():
        c0 = fg1 * CG
        pltpu.sync_copy(feat_hbm.at[b1, 0, pl.ds(c0 * N, CG * N)], feat_v)

        def qchunk(qc, _):
            pltpu.sync_copy(idx_hbm.at[b1, 0, pl.ds(qc * FLAT, FLAT)], idx_v)

            @plsc.parallel_loop(0, NVR, step=1, unroll=4)
            def _vloop(v):
                iv = idx_v[pl.ds(v * 16, 16)]
                for c in range(CG):
                    coff = jnp.full((16,), c * N, jnp.int32)
                    out_v[c, pl.ds(v * 16, 16)] = plsc.load_gather(
                        feat_v, [iv + coff])

            pltpu.sync_copy(
                out_v, ofeat_hbm.at[b1, pl.ds(c0, CG), pl.ds(qc * FLAT, FLAT)])
            return 0

        lax.fori_loop(0, NQC, qchunk, 0)

    run_feat()

    @pl.when(wid < 8)
    def _():
        b2 = wid // 2
        half = wid % 2
        pltpu.sync_copy(xyzt_hbm.at[b2, 0, :], feat_v.at[pl.ds(0, 3 * N)])

        def qchunk(qc, _):
            pltpu.sync_copy(idx_hbm.at[b2, 0, pl.ds(qc * FLAT, FLAT)], idx_v)
            pltpu.sync_copy(nq_hbm.at[b2, 0, pl.ds(qc * QC * 3, QC * 3)], nq_v)

            @plsc.parallel_loop(0, NVR, step=1, unroll=4)
            def _vloop(v):
                iv = idx_v[pl.ds(v * 16, 16)]
                for c in range(3):
                    coff = jnp.full((16,), c * N, jnp.int32)
                    g = plsc.load_gather(feat_v, [iv + coff])
                    nqi = jnp.broadcast_to((v // 2) * 3 + c, (16,))
                    g = g - plsc.load_gather(nq_v, [nqi])
                    out_v[c, pl.ds(v * 16, 16)] = g

            pltpu.sync_copy(
                out_v.at[pl.ds(0, 3), :],
                oxyz_hbm.at[b2, :, pl.ds(qc * FLAT, FLAT)])
            return 0

        lax.fori_loop(half * (NQC // 2), (half + 1) * (NQC // 2), qchunk, 0)


def _gather(xyzt, features, new_xyz, idx_flat):
    mesh = plsc.VectorSubcoreMesh(core_axis_name="c", subcore_axis_name="s",
                                  num_cores=2, num_subcores=16)
    f = pl.kernel(
        _gather_body,
        out_type=[
            jax.ShapeDtypeStruct((B, 3, NP * NSAMPLE), jnp.float32),
            jax.ShapeDtypeStruct((B, C, NP * NSAMPLE), jnp.float32),
        ],
        mesh=mesh,
        scratch_types=[
            pltpu.VMEM((CG * N,), jnp.float32),
            pltpu.VMEM((FLAT,), jnp.int32),
            pltpu.VMEM((CG, FLAT), jnp.float32),
            pltpu.VMEM((QC * 3,), jnp.float32),
        ],
        compiler_params=pltpu.CompilerParams(needs_layout_passes=False),
    )
    return f(xyzt, features, new_xyz, idx_flat)


def kernel(xyz, new_xyz, features):
    tri = jnp.triu(jnp.ones((K, K), jnp.float32), k=1)
    cnt_f, idx = _ball_query(xyz, new_xyz, tri)
    xyzt = jnp.transpose(xyz, (0, 2, 1)).reshape(B, 1, 3 * N)
    feats_flat = features.reshape(B, 1, C * N)
    nq_flat = new_xyz.reshape(B, 1, NP * 3)
    idx_flat = idx.reshape(B, 1, NP * NSAMPLE)
    oxyz, ofeat = _gather(xyzt, feats_flat, nq_flat, idx_flat)
    new_features = jnp.concatenate(
        [oxyz.reshape(B, 3, NP, NSAMPLE), ofeat.reshape(B, C, NP, NSAMPLE)],
        axis=1)
    return cnt_f.reshape(B, NP).astype(jnp.int32), new_features


# speedup vs baseline: 31.7720x; 2.3493x over previous
import functools
import jax
import jax.numpy as jnp
import numpy as np
from jax import lax
from jax.experimental import pallas as pl
from jax.experimental.pallas import tpu as pltpu
from jax.experimental.pallas import tpu_sc as plsc

RADIUS = 0.12
NSAMPLE = 32

B, N, NP, C = 4, 8192, 2048, 64
QB = 256
K = 512
NCHUNK = N // K
COUT = C + 3


NWRD = N // 32


def _ball_pack_body(xyz_ref, q_ref, plo_ref, phi_ref, cnt_ref, wrd_ref):
    q = q_ref[0]
    qq = jnp.sum(q * q, axis=1, keepdims=True)
    r2 = RADIUS * RADIUS
    plo = plo_ref[...]
    phi = phi_ref[...]

    cols = []
    cnt = jnp.zeros((QB, 1), jnp.float32)
    for k in range(NCHUNK):
        xc = xyz_ref[0, k * K:(k + 1) * K, :]
        pp = jnp.sum(xc * xc, axis=1)[None, :]
        qp = jnp.dot(q, xc.T, preferred_element_type=jnp.float32)
        d2 = qq + pp - 2.0 * qp
        m = (d2 < r2).astype(jnp.float32)
        lo = jnp.dot(m, plo, preferred_element_type=jnp.float32)
        hi = jnp.dot(m, phi, preferred_element_type=jnp.float32)
        w32 = lo.astype(jnp.int32) | (hi.astype(jnp.int32) << 16)
        cols.append(w32)
        cnt = cnt + jnp.sum(m, axis=1, keepdims=True)

    wrd_ref[0] = jnp.concatenate(cols, axis=1)
    cnt_ref[0, 0, :] = jnp.minimum(cnt, jnp.float32(NSAMPLE))[:, 0]


def _ball_pack(xyz, new_xyz, plo, phi):
    grid = (B, NP // QB)
    return pl.pallas_call(
        _ball_pack_body,
        grid=grid,
        in_specs=[
            pl.BlockSpec((1, N, 3), lambda b, q: (b, 0, 0)),
            pl.BlockSpec((1, QB, 3), lambda b, q: (b, q, 0)),
            pl.BlockSpec((K, K // 32), lambda b, q: (0, 0)),
            pl.BlockSpec((K, K // 32), lambda b, q: (0, 0)),
        ],
        out_specs=[
            pl.BlockSpec((1, 1, QB), lambda b, q: (b, 0, q)),
            pl.BlockSpec((1, QB, NWRD), lambda b, q: (b, q, 0)),
        ],
        out_shape=[
            jax.ShapeDtypeStruct((B, 1, NP), jnp.float32),
            jax.ShapeDtypeStruct((B, NP, NWRD), jnp.int32),
        ],
    )(xyz, new_xyz, plo, phi)


QT = (B * NP) // 32
QCW = 64


def _extract_body(wrd_hbm, idx_hbm, wtile_v, nzw_v, nzp_v, idxbuf_v, idxout_v):
    wid = lax.axis_index("c") * 16 + lax.axis_index("s")
    iota = lax.iota(jnp.int32, 16)
    g0 = wid * QT
    b = g0 // NP
    q0 = g0 % NP

    for ch in range(QT // QCW):
        qb = q0 + ch * QCW
        pltpu.sync_copy(wrd_hbm.at[b, pl.ds(qb, QCW), :], wtile_v)

        def per_query(j, _):
            def comp(wb, nw):
                wv = wtile_v[j, pl.ds(wb * 16, 16)]
                nz = wv != 0
                plsc.store_compressed(nzw_v.at[pl.ds(nw, 16)], wv, mask=nz)
                plsc.store_compressed(nzp_v.at[pl.ds(nw, 16)], wb * 16 + iota,
                                      mask=nz)
                return nw + plsc.all_reduce_population_count(nz)[0]

            nw = lax.fori_loop(0, NWRD // 16, comp, jnp.int32(0))
            idxbuf_v[pl.ds(0, 16)] = jnp.zeros((16,), jnp.int32)

            def cond(c):
                jw, found = c
                return (jw < nw) & (found < NSAMPLE)

            def expand(c):
                jw, found = c
                w = jnp.broadcast_to(nzw_v[pl.ds(jw, 16)][0], (16,))
                wp = nzp_v[pl.ds(jw, 16)][0]
                blo = ((w >> iota) & 1) != 0
                bhi = ((w >> (iota + 16)) & 1) != 0
                vlo = wp * 32 + iota
                plsc.store_compressed(idxbuf_v.at[pl.ds(found, 16)], vlo,
                                      mask=blo)
                found = found + plsc.all_reduce_population_count(blo)[0]
                plsc.store_compressed(idxbuf_v.at[pl.ds(found, 16)], vlo + 16,
                                      mask=bhi)
                found = found + plsc.all_reduce_population_count(bhi)[0]
                return jw + 1, found

            _, found = lax.while_loop(cond, expand,
                                      (jnp.int32(0), jnp.int32(0)))
            kf = jnp.minimum(found, NSAMPLE)
            v0 = idxbuf_v[pl.ds(0, 16)]
            v1 = idxbuf_v[pl.ds(16, 16)]
            first = jnp.broadcast_to(v0[0], (16,))
            idxout_v[pl.ds(j * 32, 16)] = jnp.where(iota < kf, v0, first)
            idxout_v[pl.ds(j * 32 + 16, 16)] = jnp.where(iota + 16 < kf, v1,
                                                         first)
            return 0

        lax.fori_loop(0, QCW, per_query, 0)
        pltpu.sync_copy(idxout_v,
                        idx_hbm.at[b, 0, pl.ds(qb * NSAMPLE, QCW * NSAMPLE)])


def _extract(words):
    mesh = plsc.VectorSubcoreMesh(core_axis_name="c", subcore_axis_name="s",
                                  num_cores=2, num_subcores=16)
    f = pl.kernel(
        _extract_body,
        out_type=jax.ShapeDtypeStruct((B, 1, NP * NSAMPLE), jnp.int32),
        mesh=mesh,
        scratch_types=[
            pltpu.VMEM((QCW, NWRD), jnp.int32),
            pltpu.VMEM((NWRD + 16,), jnp.int32),
            pltpu.VMEM((NWRD + 16,), jnp.int32),
            pltpu.VMEM((96,), jnp.int32),
            pltpu.VMEM((QCW * NSAMPLE,), jnp.int32),
        ],
        compiler_params=pltpu.CompilerParams(needs_layout_passes=False),
    )
    return f(words)


NW = 32
CG = 8
QC = 128
FLAT = QC * NSAMPLE
NVR = FLAT // 16
NQC = NP // QC


def _gather_body(xyzt_hbm, feat_hbm, nq_hbm, idx_hbm, oxyz_hbm, ofeat_hbm,
                 feat_v, idx_v, out_v, nq_v):
    wid = lax.axis_index("c") * 16 + lax.axis_index("s")

    b1 = wid // 8
    fg1 = wid % 8

    def run_feat():
        c0 = fg1 * CG
        pltpu.sync_copy(feat_hbm.at[b1, 0, pl.ds(c0 * N, CG * N)], feat_v)

        def qchunk(qc, _):
            pltpu.sync_copy(idx_hbm.at[b1, 0, pl.ds(qc * FLAT, FLAT)], idx_v)

            @plsc.parallel_loop(0, NVR, step=1, unroll=4)
            def _vloop(v):
                iv = idx_v[pl.ds(v * 16, 16)]
                for c in range(CG):
                    coff = jnp.full((16,), c * N, jnp.int32)
                    out_v[c, pl.ds(v * 16, 16)] = plsc.load_gather(
                        feat_v, [iv + coff])

            pltpu.sync_copy(
                out_v, ofeat_hbm.at[b1, pl.ds(c0, CG), pl.ds(qc * FLAT, FLAT)])
            return 0

        lax.fori_loop(0, NQC, qchunk, 0)

    run_feat()

    @pl.when(wid < 8)
    def _():
        b2 = wid // 2
        half = wid % 2
        pltpu.sync_copy(xyzt_hbm.at[b2, 0, :], feat_v.at[pl.ds(0, 3 * N)])

        def qchunk(qc, _):
            pltpu.sync_copy(idx_hbm.at[b2, 0, pl.ds(qc * FLAT, FLAT)], idx_v)
            pltpu.sync_copy(nq_hbm.at[b2, 0, pl.ds(qc * QC * 3, QC * 3)], nq_v)

            @plsc.parallel_loop(0, NVR, step=1, unroll=4)
            def _vloop(v):
                iv = idx_v[pl.ds(v * 16, 16)]
                for c in range(3):
                    coff = jnp.full((16,), c * N, jnp.int32)
                    g = plsc.load_gather(feat_v, [iv + coff])
                    nqi = jnp.broadcast_to((v // 2) * 3 + c, (16,))
                    g = g - plsc.load_gather(nq_v, [nqi])
                    out_v[c, pl.ds(v * 16, 16)] = g

            pltpu.sync_copy(
                out_v.at[pl.ds(0, 3), :],
                oxyz_hbm.at[b2, :, pl.ds(qc * FLAT, FLAT)])
            return 0

        lax.fori_loop(half * (NQC // 2), (half + 1) * (NQC // 2), qchunk, 0)


def _gather(xyzt, features, new_xyz, idx_flat):
    mesh = plsc.VectorSubcoreMesh(core_axis_name="c", subcore_axis_name="s",
                                  num_cores=2, num_subcores=16)
    f = pl.kernel(
        _gather_body,
        out_type=[
            jax.ShapeDtypeStruct((B, 3, NP * NSAMPLE), jnp.float32),
            jax.ShapeDtypeStruct((B, C, NP * NSAMPLE), jnp.float32),
        ],
        mesh=mesh,
        scratch_types=[
            pltpu.VMEM((CG * N,), jnp.float32),
            pltpu.VMEM((FLAT,), jnp.int32),
            pltpu.VMEM((CG, FLAT), jnp.float32),
            pltpu.VMEM((QC * 3,), jnp.float32),
        ],
        compiler_params=pltpu.CompilerParams(needs_layout_passes=False),
    )
    return f(xyzt, features, new_xyz, idx_flat)


def _pack_mats():
    j = np.arange(K)
    u = np.arange(K // 32)
    sel = (j[:, None] // 32) == u[None, :]
    bit = j % 32
    plo = np.where(sel & (bit[:, None] < 16), 2.0 ** (bit[:, None]), 0.0)
    phi = np.where(sel & (bit[:, None] >= 16), 2.0 ** (bit[:, None] - 16), 0.0)
    return (jnp.asarray(plo, jnp.float32), jnp.asarray(phi, jnp.float32))


def kernel(xyz, new_xyz, features):
    plo, phi = _pack_mats()
    cnt_f, words = _ball_pack(xyz, new_xyz, plo, phi)
    idx_flat = _extract(words)
    xyzt = jnp.transpose(xyz, (0, 2, 1)).reshape(B, 1, 3 * N)
    feats_flat = features.reshape(B, 1, C * N)
    nq_flat = new_xyz.reshape(B, 1, NP * 3)
    oxyz, ofeat = _gather(xyzt, feats_flat, nq_flat, idx_flat)
    new_features = jnp.concatenate(
        [oxyz.reshape(B, 3, NP, NSAMPLE), ofeat.reshape(B, C, NP, NSAMPLE)],
        axis=1)
    return cnt_f.reshape(B, NP).astype(jnp.int32), new_features


# speedup vs baseline: 31.9313x; 1.0050x over previous
import functools
import jax
import jax.numpy as jnp
import numpy as np
from jax import lax
from jax.experimental import pallas as pl
from jax.experimental.pallas import tpu as pltpu
from jax.experimental.pallas import tpu_sc as plsc

RADIUS = 0.12
NSAMPLE = 32

B, N, NP, C = 4, 8192, 2048, 64
QB = 256
K = 512
NCHUNK = N // K
COUT = C + 3


NWRD = N // 32


def _ball_pack_body(xyz_ref, q_ref, plo_ref, phi_ref, cnt_ref, wrd_ref):
    q = q_ref[0]
    qq = jnp.sum(q * q, axis=1, keepdims=True)
    r2 = RADIUS * RADIUS
    plo = plo_ref[...]
    phi = phi_ref[...]

    cols = []
    cnt = jnp.zeros((QB, 1), jnp.float32)
    for k in range(NCHUNK):
        xc = xyz_ref[0, k * K:(k + 1) * K, :]
        pp = jnp.sum(xc * xc, axis=1)[None, :]
        qp = jnp.dot(q, xc.T, preferred_element_type=jnp.float32)
        d2 = qq + pp - 2.0 * qp
        m = (d2 < r2).astype(jnp.float32)
        lo = jnp.dot(m, plo, preferred_element_type=jnp.float32)
        hi = jnp.dot(m, phi, preferred_element_type=jnp.float32)
        w32 = lo.astype(jnp.int32) | (hi.astype(jnp.int32) << 16)
        cols.append(w32)
        cnt = cnt + jnp.sum(m, axis=1, keepdims=True)

    wrd_ref[0] = jnp.concatenate(cols, axis=1)
    cnt_ref[0, 0, :] = jnp.minimum(cnt, jnp.float32(NSAMPLE))[:, 0]


def _ball_pack(xyz, new_xyz, plo, phi):
    grid = (B, NP // QB)
    return pl.pallas_call(
        _ball_pack_body,
        grid=grid,
        in_specs=[
            pl.BlockSpec((1, N, 3), lambda b, q: (b, 0, 0)),
            pl.BlockSpec((1, QB, 3), lambda b, q: (b, q, 0)),
            pl.BlockSpec((K, K // 32), lambda b, q: (0, 0)),
            pl.BlockSpec((K, K // 32), lambda b, q: (0, 0)),
        ],
        out_specs=[
            pl.BlockSpec((1, 1, QB), lambda b, q: (b, 0, q)),
            pl.BlockSpec((1, QB, NWRD), lambda b, q: (b, q, 0)),
        ],
        out_shape=[
            jax.ShapeDtypeStruct((B, 1, NP), jnp.float32),
            jax.ShapeDtypeStruct((B, NP, NWRD), jnp.int32),
        ],
    )(xyz, new_xyz, plo, phi)


QT = (B * NP) // 32
QCW = 64


def _extract_body(wrd_hbm, idx_hbm, wtile_v, nzw_v, nzp_v, idxbuf_v, idxout_v):
    wid = lax.axis_index("c") * 16 + lax.axis_index("s")
    iota = lax.iota(jnp.int32, 16)
    g0 = wid * QT
    b = g0 // NP
    q0 = g0 % NP

    for ch in range(QT // QCW):
        qb = q0 + ch * QCW
        pltpu.sync_copy(wrd_hbm.at[b, pl.ds(qb, QCW), :], wtile_v)

        def per_query(j, _):
            def comp(wb, nw):
                wv = wtile_v[j, pl.ds(wb * 16, 16)]
                nz = wv != 0
                plsc.store_compressed(nzw_v.at[pl.ds(nw, 16)], wv, mask=nz)
                plsc.store_compressed(nzp_v.at[pl.ds(nw, 16)], wb * 16 + iota,
                                      mask=nz)
                return nw + plsc.all_reduce_population_count(nz)[0]

            nw = lax.fori_loop(0, NWRD // 16, comp, jnp.int32(0), unroll=4)
            idxbuf_v[pl.ds(0, 16)] = jnp.zeros((16,), jnp.int32)

            def cond(c):
                jw, found = c
                return (jw < nw) & (found < NSAMPLE)

            def expand(c):
                jw, found = c
                w = jnp.broadcast_to(nzw_v[pl.ds(jw, 16)][0], (16,))
                wp = nzp_v[pl.ds(jw, 16)][0]
                blo = ((w >> iota) & 1) != 0
                bhi = ((w >> (iota + 16)) & 1) != 0
                vlo = wp * 32 + iota
                plsc.store_compressed(idxbuf_v.at[pl.ds(found, 16)], vlo,
                                      mask=blo)
                found = found + plsc.all_reduce_population_count(blo)[0]
                plsc.store_compressed(idxbuf_v.at[pl.ds(found, 16)], vlo + 16,
                                      mask=bhi)
                found = found + plsc.all_reduce_population_count(bhi)[0]
                return jw + 1, found

            _, found = lax.while_loop(cond, expand,
                                      (jnp.int32(0), jnp.int32(0)))
            kf = jnp.minimum(found, NSAMPLE)
            v0 = idxbuf_v[pl.ds(0, 16)]
            v1 = idxbuf_v[pl.ds(16, 16)]
            first = jnp.broadcast_to(v0[0], (16,))
            idxout_v[pl.ds(j * 32, 16)] = jnp.where(iota < kf, v0, first)
            idxout_v[pl.ds(j * 32 + 16, 16)] = jnp.where(iota + 16 < kf, v1,
                                                         first)
            return 0

        lax.fori_loop(0, QCW, per_query, 0)
        pltpu.sync_copy(idxout_v,
                        idx_hbm.at[b, 0, pl.ds(qb * NSAMPLE, QCW * NSAMPLE)])


def _extract(words):
    mesh = plsc.VectorSubcoreMesh(core_axis_name="c", subcore_axis_name="s",
                                  num_cores=2, num_subcores=16)
    f = pl.kernel(
        _extract_body,
        out_type=jax.ShapeDtypeStruct((B, 1, NP * NSAMPLE), jnp.int32),
        mesh=mesh,
        scratch_types=[
            pltpu.VMEM((QCW, NWRD), jnp.int32),
            pltpu.VMEM((NWRD + 16,), jnp.int32),
            pltpu.VMEM((NWRD + 16,), jnp.int32),
            pltpu.VMEM((96,), jnp.int32),
            pltpu.VMEM((QCW * NSAMPLE,), jnp.int32),
        ],
        compiler_params=pltpu.CompilerParams(needs_layout_passes=False),
    )
    return f(words)


NW = 32
CG = 8
QC = 128
FLAT = QC * NSAMPLE
NVR = FLAT // 16
NQC = NP // QC


def _gather_body(xyzt_hbm, feat_hbm, nq_hbm, idx_hbm, oxyz_hbm, ofeat_hbm,
                 feat_v, idx_v, out_v, nq_v, sem):
    wid = lax.axis_index("c") * 16 + lax.axis_index("s")

    b1 = wid // 8
    fg1 = wid % 8

    def run_feat():
        c0 = fg1 * CG
        pltpu.sync_copy(feat_hbm.at[b1, 0, pl.ds(c0 * N, CG * N)], feat_v)

        def qchunk(qc, _):
            pltpu.sync_copy(idx_hbm.at[b1, 0, pl.ds(qc * FLAT, FLAT)], idx_v)

            @plsc.parallel_loop(0, NVR, step=1, unroll=4)
            def _vloop(v):
                iv = idx_v[pl.ds(v * 16, 16)]
                for c in range(CG):
                    coff = jnp.full((16,), c * N, jnp.int32)
                    out_v[c, pl.ds(v * 16, 16)] = plsc.load_gather(
                        feat_v, [iv + coff])

            pltpu.sync_copy(
                out_v, ofeat_hbm.at[b1, pl.ds(c0, CG), pl.ds(qc * FLAT, FLAT)])
            return 0

        lax.fori_loop(0, NQC, qchunk, 0)

    run_feat()

    @pl.when((wid % 16) < 4)
    def _():
        xid = (wid // 16) * 4 + (wid % 16)
        b2 = xid // 2
        half = xid % 2
        pltpu.sync_copy(xyzt_hbm.at[b2, 0, :], feat_v.at[pl.ds(0, 3 * N)])

        def qchunk(qc, _):
            pltpu.sync_copy(idx_hbm.at[b2, 0, pl.ds(qc * FLAT, FLAT)], idx_v)
            pltpu.sync_copy(nq_hbm.at[b2, 0, pl.ds(qc * QC * 3, QC * 3)], nq_v)

            @plsc.parallel_loop(0, NVR, step=1, unroll=4)
            def _vloop(v):
                iv = idx_v[pl.ds(v * 16, 16)]
                for c in range(3):
                    coff = jnp.full((16,), c * N, jnp.int32)
                    g = plsc.load_gather(feat_v, [iv + coff])
                    nqi = jnp.broadcast_to((v // 2) * 3 + c, (16,))
                    g = g - plsc.load_gather(nq_v, [nqi])
                    out_v[c, pl.ds(v * 16, 16)] = g

            pltpu.sync_copy(
                out_v.at[pl.ds(0, 3), :],
                oxyz_hbm.at[b2, :, pl.ds(qc * FLAT, FLAT)])
            return 0

        lax.fori_loop(half * (NQC // 2), (half + 1) * (NQC // 2), qchunk, 0)


def _gather(xyzt, features, new_xyz, idx_flat):
    mesh = plsc.VectorSubcoreMesh(core_axis_name="c", subcore_axis_name="s",
                                  num_cores=2, num_subcores=16)
    f = pl.kernel(
        _gather_body,
        out_type=[
            jax.ShapeDtypeStruct((B, 3, NP * NSAMPLE), jnp.float32),
            jax.ShapeDtypeStruct((B, C, NP * NSAMPLE), jnp.float32),
        ],
        mesh=mesh,
        scratch_types=[
            pltpu.VMEM((CG * N,), jnp.float32),
            pltpu.VMEM((FLAT,), jnp.int32),
            pltpu.VMEM((CG, FLAT), jnp.float32),
            pltpu.VMEM((QC * 3,), jnp.float32),
            pltpu.SemaphoreType.DMA,
        ],
        compiler_params=pltpu.CompilerParams(needs_layout_passes=False),
    )
    return f(xyzt, features, new_xyz, idx_flat)


def _pack_mats():
    j = np.arange(K)
    u = np.arange(K // 32)
    sel = (j[:, None] // 32) == u[None, :]
    bit = j % 32
    plo = np.where(sel & (bit[:, None] < 16), 2.0 ** (bit[:, None]), 0.0)
    phi = np.where(sel & (bit[:, None] >= 16), 2.0 ** (bit[:, None] - 16), 0.0)
    return (jnp.asarray(plo, jnp.float32), jnp.asarray(phi, jnp.float32))


def kernel(xyz, new_xyz, features):
    plo, phi = _pack_mats()
    cnt_f, words = _ball_pack(xyz, new_xyz, plo, phi)
    idx_flat = _extract(words)
    xyzt = jnp.transpose(xyz, (0, 2, 1)).reshape(B, 1, 3 * N)
    feats_flat = features.reshape(B, 1, C * N)
    nq_flat = new_xyz.reshape(B, 1, NP * 3)
    oxyz, ofeat = _gather(xyzt, feats_flat, nq_flat, idx_flat)
    new_features = jnp.concatenate(
        [oxyz.reshape(B, 3, NP, NSAMPLE), ofeat.reshape(B, C, NP, NSAMPLE)],
        axis=1)
    return cnt_f.reshape(B, NP).astype(jnp.int32), new_features


# speedup vs baseline: 32.6479x; 1.0224x over previous
import functools
import jax
import jax.numpy as jnp
import numpy as np
from jax import lax
from jax.experimental import pallas as pl
from jax.experimental.pallas import tpu as pltpu
from jax.experimental.pallas import tpu_sc as plsc

RADIUS = 0.12
NSAMPLE = 32

B, N, NP, C = 4, 8192, 2048, 64
QB = 256
K = 512
NCHUNK = N // K
COUT = C + 3


NWRD = N // 32


def _ball_pack_body(x5_ref, q5_ref, plo_ref, phi_ref, cnt_ref, wrd_ref):
    q5 = q5_ref[0]
    r2 = RADIUS * RADIUS
    plo = plo_ref[...]
    phi = phi_ref[...]
    ones = jnp.ones((K, 1), jnp.float32)

    cols = []
    cnt = jnp.zeros((QB, 1), jnp.float32)
    for k in range(NCHUNK):
        x5c = x5_ref[0, k * K:(k + 1) * K, :]
        d2 = jnp.dot(q5, x5c.T, preferred_element_type=jnp.float32)
        m = (d2 < r2).astype(jnp.float32)
        lo = jnp.dot(m, plo, preferred_element_type=jnp.float32)
        hi = jnp.dot(m, phi, preferred_element_type=jnp.float32)
        w32 = lo.astype(jnp.int32) | (hi.astype(jnp.int32) << 16)
        cols.append(w32)
        cnt = cnt + jnp.dot(m, ones, preferred_element_type=jnp.float32)

    wrd_ref[0] = jnp.concatenate(cols, axis=1)
    cnt_ref[0, 0, :] = jnp.minimum(cnt, jnp.float32(NSAMPLE))[:, 0]


def _ball_pack(x5, q5, plo, phi):
    grid = (B, NP // QB)
    return pl.pallas_call(
        _ball_pack_body,
        grid=grid,
        in_specs=[
            pl.BlockSpec((1, N, 5), lambda b, q: (b, 0, 0)),
            pl.BlockSpec((1, QB, 5), lambda b, q: (b, q, 0)),
            pl.BlockSpec((K, K // 32), lambda b, q: (0, 0)),
            pl.BlockSpec((K, K // 32), lambda b, q: (0, 0)),
        ],
        out_specs=[
            pl.BlockSpec((1, 1, QB), lambda b, q: (b, 0, q)),
            pl.BlockSpec((1, QB, NWRD), lambda b, q: (b, q, 0)),
        ],
        out_shape=[
            jax.ShapeDtypeStruct((B, 1, NP), jnp.float32),
            jax.ShapeDtypeStruct((B, NP, NWRD), jnp.int32),
        ],
    )(x5, q5, plo, phi)


QT = (B * NP) // 32
QCW = 64


def _extract_body(wrd_hbm, idx_hbm, wtile_v, nzw_v, nzp_v, idxbuf_v, idxout_v):
    wid = lax.axis_index("c") * 16 + lax.axis_index("s")
    iota = lax.iota(jnp.int32, 16)
    g0 = wid * QT
    b = g0 // NP
    q0 = g0 % NP

    for ch in range(QT // QCW):
        qb = q0 + ch * QCW
        pltpu.sync_copy(wrd_hbm.at[b, pl.ds(qb, QCW), :], wtile_v)

        def per_query(j, _):
            def comp(wb, nw):
                wv = wtile_v[j, pl.ds(wb * 16, 16)]
                nz = wv != 0
                plsc.store_compressed(nzw_v.at[pl.ds(nw, 16)], wv, mask=nz)
                plsc.store_compressed(nzp_v.at[pl.ds(nw, 16)], wb * 16 + iota,
                                      mask=nz)
                return nw + plsc.all_reduce_population_count(nz)[0]

            nw = lax.fori_loop(0, NWRD // 16, comp, jnp.int32(0), unroll=4)
            idxbuf_v[pl.ds(0, 16)] = jnp.zeros((16,), jnp.int32)

            def cond(c):
                jw, found = c
                return (jw < nw) & (found < NSAMPLE)

            def expand(c):
                jw, found = c
                w = jnp.broadcast_to(nzw_v[pl.ds(jw, 16)][0], (16,))
                wp = nzp_v[pl.ds(jw, 16)][0]
                blo = ((w >> iota) & 1) != 0
                bhi = ((w >> (iota + 16)) & 1) != 0
                vlo = wp * 32 + iota
                plsc.store_compressed(idxbuf_v.at[pl.ds(found, 16)], vlo,
                                      mask=blo)
                found = found + plsc.all_reduce_population_count(blo)[0]
                plsc.store_compressed(idxbuf_v.at[pl.ds(found, 16)], vlo + 16,
                                      mask=bhi)
                found = found + plsc.all_reduce_population_count(bhi)[0]
                return jw + 1, found

            _, found = lax.while_loop(cond, expand,
                                      (jnp.int32(0), jnp.int32(0)))
            kf = jnp.minimum(found, NSAMPLE)
            v0 = idxbuf_v[pl.ds(0, 16)]
            v1 = idxbuf_v[pl.ds(16, 16)]
            first = jnp.broadcast_to(v0[0], (16,))
            idxout_v[pl.ds(j * 32, 16)] = jnp.where(iota < kf, v0, first)
            idxout_v[pl.ds(j * 32 + 16, 16)] = jnp.where(iota + 16 < kf, v1,
                                                         first)
            return 0

        lax.fori_loop(0, QCW, per_query, 0)
        pltpu.sync_copy(idxout_v,
                        idx_hbm.at[b, 0, pl.ds(qb * NSAMPLE, QCW * NSAMPLE)])


def _extract(words):
    mesh = plsc.VectorSubcoreMesh(core_axis_name="c", subcore_axis_name="s",
                                  num_cores=2, num_subcores=16)
    f = pl.kernel(
        _extract_body,
        out_type=jax.ShapeDtypeStruct((B, 1, NP * NSAMPLE), jnp.int32),
        mesh=mesh,
        scratch_types=[
            pltpu.VMEM((QCW, NWRD), jnp.int32),
            pltpu.VMEM((NWRD + 16,), jnp.int32),
            pltpu.VMEM((NWRD + 16,), jnp.int32),
            pltpu.VMEM((96,), jnp.int32),
            pltpu.VMEM((QCW * NSAMPLE,), jnp.int32),
        ],
        compiler_params=pltpu.CompilerParams(needs_layout_passes=False),
    )
    return f(words)


NW = 32
CG = 8
QC = 128
FLAT = QC * NSAMPLE
NVR = FLAT // 16
NQC = NP // QC


def _gather_body(xyzt_hbm, feat_hbm, nq_hbm, idx_hbm, oxyz_hbm, ofeat_hbm,
                 feat_v, idx_v, out_v, nq_v, sem):
    wid = lax.axis_index("c") * 16 + lax.axis_index("s")

    b1 = wid // 8
    fg1 = wid % 8

    def run_feat():
        c0 = fg1 * CG
        pltpu.sync_copy(feat_hbm.at[b1, 0, pl.ds(c0 * N, CG * N)], feat_v)

        def qchunk(qc, _):
            pltpu.sync_copy(idx_hbm.at[b1, 0, pl.ds(qc * FLAT, FLAT)], idx_v)

            @plsc.parallel_loop(0, NVR, step=1, unroll=4)
            def _vloop(v):
                iv = idx_v[pl.ds(v * 16, 16)]
                for c in range(CG):
                    coff = jnp.full((16,), c * N, jnp.int32)
                    out_v[c, pl.ds(v * 16, 16)] = plsc.load_gather(
                        feat_v, [iv + coff])

            pltpu.sync_copy(
                out_v, ofeat_hbm.at[b1, pl.ds(c0, CG), pl.ds(qc * FLAT, FLAT)])
            return 0

        lax.fori_loop(0, NQC, qchunk, 0)

    run_feat()

    @pl.when((wid % 16) < 4)
    def _():
        xid = (wid // 16) * 4 + (wid % 16)
        b2 = xid // 2
        half = xid % 2
        pltpu.sync_copy(xyzt_hbm.at[b2, 0, :], feat_v.at[pl.ds(0, 3 * N)])

        def qchunk(qc, _):
            pltpu.sync_copy(idx_hbm.at[b2, 0, pl.ds(qc * FLAT, FLAT)], idx_v)
            pltpu.sync_copy(nq_hbm.at[b2, 0, pl.ds(qc * QC * 3, QC * 3)], nq_v)

            @plsc.parallel_loop(0, NVR, step=1, unroll=4)
            def _vloop(v):
                iv = idx_v[pl.ds(v * 16, 16)]
                for c in range(3):
                    coff = jnp.full((16,), c * N, jnp.int32)
                    g = plsc.load_gather(feat_v, [iv + coff])
                    nqi = jnp.broadcast_to((v // 2) * 3 + c, (16,))
                    g = g - plsc.load_gather(nq_v, [nqi])
                    out_v[c, pl.ds(v * 16, 16)] = g

            pltpu.sync_copy(
                out_v.at[pl.ds(0, 3), :],
                oxyz_hbm.at[b2, :, pl.ds(qc * FLAT, FLAT)])
            return 0

        lax.fori_loop(half * (NQC // 2), (half + 1) * (NQC // 2), qchunk, 0)


def _gather(xyzt, features, new_xyz, idx_flat):
    mesh = plsc.VectorSubcoreMesh(core_axis_name="c", subcore_axis_name="s",
                                  num_cores=2, num_subcores=16)
    f = pl.kernel(
        _gather_body,
        out_type=[
            jax.ShapeDtypeStruct((B, 3, NP * NSAMPLE), jnp.float32),
            jax.ShapeDtypeStruct((B, C, NP * NSAMPLE), jnp.float32),
        ],
        mesh=mesh,
        scratch_types=[
            pltpu.VMEM((CG * N,), jnp.float32),
            pltpu.VMEM((FLAT,), jnp.int32),
            pltpu.VMEM((CG, FLAT), jnp.float32),
            pltpu.VMEM((QC * 3,), jnp.float32),
            pltpu.SemaphoreType.DMA,
        ],
        compiler_params=pltpu.CompilerParams(needs_layout_passes=False),
    )
    return f(xyzt, features, new_xyz, idx_flat)


def _pack_mats():
    j = np.arange(K)
    u = np.arange(K // 32)
    sel = (j[:, None] // 32) == u[None, :]
    bit = j % 32
    plo = np.where(sel & (bit[:, None] < 16), 2.0 ** (bit[:, None]), 0.0)
    phi = np.where(sel & (bit[:, None] >= 16), 2.0 ** (bit[:, None] - 16), 0.0)
    return (jnp.asarray(plo, jnp.float32), jnp.asarray(phi, jnp.float32))


def kernel(xyz, new_xyz, features):
    plo, phi = _pack_mats()
    pp = jnp.sum(xyz * xyz, axis=-1, keepdims=True)
    qq = jnp.sum(new_xyz * new_xyz, axis=-1, keepdims=True)
    one_n = jnp.ones((B, N, 1), jnp.float32)
    one_q = jnp.ones((B, NP, 1), jnp.float32)
    x5 = jnp.concatenate([xyz, pp, one_n], axis=-1)
    q5 = jnp.concatenate([-2.0 * new_xyz, one_q, qq], axis=-1)
    cnt_f, words = _ball_pack(x5, q5, plo, phi)
    idx_flat = _extract(words)
    xyzt = jnp.transpose(xyz, (0, 2, 1)).reshape(B, 1, 3 * N)
    feats_flat = features.reshape(B, 1, C * N)
    nq_flat = new_xyz.reshape(B, 1, NP * 3)
    oxyz, ofeat = _gather(xyzt, feats_flat, nq_flat, idx_flat)
    new_features = jnp.concatenate(
        [oxyz.reshape(B, 3, NP, NSAMPLE), ofeat.reshape(B, C, NP, NSAMPLE)],
        axis=1)
    return cnt_f.reshape(B, NP).astype(jnp.int32), new_features
